# baseline (speedup 1.0000x reference)
import jax
import jax.numpy as jnp
from jax import lax
from jax.experimental import pallas as pl
from jax.experimental.pallas import tpu as pltpu
from jax.experimental.pallas import tpu_sc as plsc

_B = 4096
_N_SPARSE = 26
_N_DENSE = 13
_VOCAB = 100000
_NC = 2
_NS = 16
_NW = _NC * _NS
_RPW = _B // _NW
_L = 16
_NF = _N_SPARSE + _N_DENSE


def _sc_body(x_hbm, table_hbm, wd_hbm, out_hbm,
             x_v, idx_v, rows_v, wd_v, acc_v, sem):
    wid = lax.axis_index("s") * _NC + lax.axis_index("c")
    base = wid * _RPW

    pltpu.sync_copy(x_hbm.at[pl.ds(base * _NF, _RPW * _NF)], x_v)
    pltpu.sync_copy(wd_hbm, wd_v)
    wdv = wd_v[:]

    lane = lax.iota(jnp.int32, _L)

    def bf16_round(v):
        u = lax.bitcast_convert_type(v, jnp.int32)
        r = (u + 0x7FFF + ((u >> 16) & 1)) & jnp.int32(-65536)
        return lax.bitcast_convert_type(r, jnp.float32)

    for j in range(_RPW // _L):
        sl = pl.ds(j * _L, _L)
        rowbase = lane * _NF + (j * _L * _NF)
        acc = jnp.zeros((_L,), jnp.float32)
        for d in range(_N_DENSE):
            acc = acc + bf16_round(plsc.load_gather(
                x_v, [rowbase + (_N_SPARSE + d)])) * wdv[d]
        acc_v[sl] = acc
        for f in range(_N_SPARSE):
            vals = plsc.load_gather(x_v, [rowbase + f])
            idx_v[f, sl] = vals.astype(jnp.int32) + (f * _VOCAB)

    copies = [
        pltpu.make_async_copy(table_hbm.at[idx_v.at[f]], rows_v.at[f], sem)
        for f in range(_N_SPARSE)
    ]
    for cp in copies:
        cp.start()
    for cp in copies:
        cp.wait()

    for j in range(_RPW // _L):
        sl = pl.ds(j * _L, _L)
        acc = acc_v[sl]
        for f in range(_N_SPARSE):
            acc = acc + rows_v[f, sl]
        acc_v[sl] = acc

    pltpu.sync_copy(acc_v, out_hbm.at[pl.ds(base, _RPW)])


@jax.jit
def _run(x, table, wd):
    mesh = plsc.VectorSubcoreMesh(core_axis_name="c", subcore_axis_name="s")
    return pl.kernel(
        _sc_body,
        out_type=jax.ShapeDtypeStruct((_B,), jnp.float32),
        mesh=mesh,
        compiler_params=pltpu.CompilerParams(needs_layout_passes=False),
        scratch_types=[
            pltpu.VMEM((_RPW * _NF,), jnp.float32),
            pltpu.VMEM((_N_SPARSE, _RPW), jnp.int32),
            pltpu.VMEM((_N_SPARSE, _RPW), jnp.float32),
            pltpu.VMEM((_L,), jnp.float32),
            pltpu.VMEM((_RPW,), jnp.float32),
            pltpu.SemaphoreType.DMA,
        ],
    )(x, table, wd)


def kernel(X, W_emb, W_dense):
    table = W_emb.reshape(-1)
    X = X.reshape(-1)
    wd = jnp.pad(W_dense[:, 0], (0, _L - _N_DENSE))
    wd = wd.astype(jnp.bfloat16).astype(jnp.float32)
    out = _run(X, table, wd)
    return out.reshape(_B, 1)
